# Initial kernel scaffold; baseline (speedup 1.0000x reference)
import jax
import jax.numpy as jnp
from jax.experimental import pallas as pl


def kernel(x, conv_weight):
    raise NotImplementedError("write your pallas kernel here")


# speedup vs baseline: 1.0544x; 1.0544x over previous
import functools

import jax
import jax.numpy as jnp
from jax.experimental import pallas as pl
from jax.experimental.pallas import tpu as pltpu


_VMEM_BUDGET = 48 * 1024 * 1024


def _eca_slab_kernel(w_ref, x_ref, o_ref, *, k):
    x = x_ref[...]
    bb, c, _ = x.shape
    s = jnp.sum(x, axis=-1, keepdims=True, dtype=jnp.float32)
    pad = (k - 1) // 2
    if pad:
        z = jnp.zeros((bb, pad, 1), jnp.float32)
        s = jnp.concatenate([z, s, z], axis=1)
    g = w_ref[0] * s[:, 0:c, :]
    for j in range(1, k):
        g = g + w_ref[j] * s[:, j:j + c, :]
    gate = jax.nn.sigmoid(g)
    o_ref[...] = (x * gate.astype(x.dtype)).astype(o_ref.dtype)


def _pick_batch_block(b, c, hw, itemsize):
    for bb in (8, 4, 2, 1):
        if b % bb == 0 and 4 * bb * c * hw * itemsize + (2 << 20) <= _VMEM_BUDGET:
            return bb
    return 1


def kernel(x, conv_weight):
    b, c, h, w = x.shape
    hw = h * w
    k = int(conv_weight.shape[0])
    itemsize = jnp.dtype(x.dtype).itemsize

    x_r = x.reshape(b, c, hw)
    taps = conv_weight.astype(jnp.float32) * jnp.float32(1.0 / hw)

    bb = _pick_batch_block(b, c, hw, itemsize)
    out_r = pl.pallas_call(
        functools.partial(_eca_slab_kernel, k=k),
        out_shape=jax.ShapeDtypeStruct((b, c, hw), x.dtype),
        grid=(b // bb,),
        in_specs=[
            pl.BlockSpec(memory_space=pltpu.MemorySpace.SMEM),
            pl.BlockSpec((bb, c, hw), lambda i: (i, 0, 0)),
        ],
        out_specs=pl.BlockSpec((bb, c, hw), lambda i: (i, 0, 0)),
        compiler_params=pltpu.CompilerParams(
            dimension_semantics=("parallel",),
            vmem_limit_bytes=_VMEM_BUDGET),
        cost_estimate=pl.CostEstimate(
            flops=int(2 * b * c * hw),
            transcendentals=int(b * c),
            bytes_accessed=int(2 * b * c * hw * itemsize)),
    )(taps, x_r)

    return out_r.reshape(b, c, h, w)


# speedup vs baseline: 4.7472x; 4.5024x over previous
import functools

import jax
import jax.numpy as jnp
from jax.experimental import pallas as pl
from jax.experimental.pallas import tpu as pltpu


_VMEM_LIMIT = 60000 * 1024


def _shift_lanes(s, d):
    bb, c = s.shape
    if d == 0:
        return s
    z = jnp.zeros((bb, abs(d)), s.dtype)
    if d > 0:
        return jnp.concatenate([s[:, d:], z], axis=-1)
    return jnp.concatenate([z, s[:, :c + d]], axis=-1)


def _eca_hw_major_kernel(w_ref, x_ref, o_ref, *, k):
    x = x_ref[...]
    pad = (k - 1) // 2
    s = jnp.sum(x, axis=0, dtype=jnp.float32)
    g = w_ref[0] * _shift_lanes(s, 0 - pad)
    for j in range(1, k):
        g = g + w_ref[j] * _shift_lanes(s, j - pad)
    gate = jax.nn.sigmoid(g)
    o_ref[...] = (x * gate[None, :, :].astype(x.dtype)).astype(o_ref.dtype)


def _eca_slab_kernel(w_ref, x_ref, o_ref, *, k):
    x = x_ref[...]
    bb, c, _ = x.shape
    s = jnp.sum(x, axis=-1, keepdims=True, dtype=jnp.float32)
    pad = (k - 1) // 2
    if pad:
        z = jnp.zeros((bb, pad, 1), jnp.float32)
        s = jnp.concatenate([z, s, z], axis=1)
    g = w_ref[0] * s[:, 0:c, :]
    for j in range(1, k):
        g = g + w_ref[j] * s[:, j:j + c, :]
    gate = jax.nn.sigmoid(g)
    o_ref[...] = (x * gate.astype(x.dtype)).astype(o_ref.dtype)


def kernel(x, conv_weight):
    b, c, h, w = x.shape
    hw = h * w
    k = int(conv_weight.shape[0])
    itemsize = jnp.dtype(x.dtype).itemsize
    taps = conv_weight.astype(jnp.float32) * jnp.float32(1.0 / hw)
    cost = pl.CostEstimate(
        flops=int(2 * b * c * hw),
        transcendentals=int(b * c),
        bytes_accessed=int(2 * b * c * hw * itemsize))

    bb = 0
    for cand in (8, 16, 24, 32):
        if b % cand == 0 and 4 * hw * cand * c * itemsize <= _VMEM_LIMIT - (7 << 20):
            bb = cand
    if bb:
        x_t = jnp.transpose(x, (2, 3, 0, 1)).reshape(hw, b, c)
        out_t = pl.pallas_call(
            functools.partial(_eca_hw_major_kernel, k=k),
            out_shape=jax.ShapeDtypeStruct((hw, b, c), x.dtype),
            grid=(b // bb,),
            in_specs=[
                pl.BlockSpec(memory_space=pltpu.MemorySpace.SMEM),
                pl.BlockSpec((hw, bb, c), lambda i: (0, i, 0)),
            ],
            out_specs=pl.BlockSpec((hw, bb, c), lambda i: (0, i, 0)),
            compiler_params=pltpu.CompilerParams(
                dimension_semantics=("parallel",),
                vmem_limit_bytes=_VMEM_LIMIT),
            cost_estimate=cost,
        )(taps, x_t)
        return out_t.reshape(h, w, b, c).transpose(2, 3, 0, 1)

    bb = 4 if b % 4 == 0 else (2 if b % 2 == 0 else 1)
    x_r = x.reshape(b, c, hw)
    out_r = pl.pallas_call(
        functools.partial(_eca_slab_kernel, k=k),
        out_shape=jax.ShapeDtypeStruct((b, c, hw), x.dtype),
        grid=(b // bb,),
        in_specs=[
            pl.BlockSpec(memory_space=pltpu.MemorySpace.SMEM),
            pl.BlockSpec((bb, c, hw), lambda i: (i, 0, 0)),
        ],
        out_specs=pl.BlockSpec((bb, c, hw), lambda i: (i, 0, 0)),
        compiler_params=pltpu.CompilerParams(
            dimension_semantics=("parallel",),
            vmem_limit_bytes=_VMEM_LIMIT),
        cost_estimate=cost,
    )(taps, x_r)
    return out_r.reshape(b, c, h, w)
